# baseline (speedup 1.0000x reference)
import functools

import jax
import jax.numpy as jnp
from jax import lax
from jax.experimental import pallas as pl
from jax.experimental.pallas import tpu as pltpu
from jax.experimental.pallas import tpu_sc as plsc

NUM_ITEMS_ = 1000000
BATCH = 16384
DIM = 32
LINE = 128
NUM_CORES = 2
NUM_SUBCORES = 16
NUM_WORKERS = NUM_CORES * NUM_SUBCORES
B_PER_W = BATCH // NUM_WORKERS
CHUNK = 128
N_CHUNKS = B_PER_W // CHUNK
LANES = 16


def _sc_body(theta_hbm, u_hbm, v_hbm, out_hbm,
             idx_u, idx_v, rows_u, rows_v, acc_v, sem):
    wid = lax.axis_index("s") * NUM_CORES + lax.axis_index("c")
    base = wid * B_PER_W

    pltpu.sync_copy(u_hbm.at[wid], idx_u)
    pltpu.sync_copy(v_hbm.at[wid], idx_v)

    def fire(c, buf):
        return [
            pltpu.async_copy(theta_hbm.at[idx_u.at[c]], rows_u.at[buf], sem),
            pltpu.async_copy(theta_hbm.at[idx_v.at[c]], rows_v.at[buf], sem),
        ]

    lane = lax.iota(jnp.int32, LANES)
    pending = fire(0, 0)
    for c in range(N_CHUNKS):
        nxt = fire(c + 1, (c + 1) % 2) if c + 1 < N_CHUNKS else []
        for cp in pending:
            cp.wait()
        pending = nxt
        bvec = jnp.full((LANES,), c % 2, jnp.int32)

        def grp(g, carry, c=c, bvec=bvec):
            rvec = g * LANES + lane
            d0 = jnp.zeros((LANES,), jnp.int32)
            acc = (plsc.load_gather(rows_u, [bvec, rvec, d0]) *
                   plsc.load_gather(rows_v, [bvec, rvec, d0]))
            for d in range(1, DIM):
                dv = jnp.full((LANES,), d, jnp.int32)
                acc = acc - (plsc.load_gather(rows_u, [bvec, rvec, dv]) *
                             plsc.load_gather(rows_v, [bvec, rvec, dv]))
            acc_v[pl.ds(c * CHUNK + g * LANES, LANES)] = acc
            return carry

        lax.fori_loop(0, CHUNK // LANES, grp, 0)

    pltpu.sync_copy(acc_v, out_hbm.at[pl.ds(base, B_PER_W)])


def _sc_lorentz(theta128, u3, v3):
    mesh = plsc.VectorSubcoreMesh(core_axis_name="c", subcore_axis_name="s")
    k = pl.kernel(
        _sc_body,
        out_type=jax.ShapeDtypeStruct((BATCH,), jnp.float32),
        mesh=mesh,
        compiler_params=pltpu.CompilerParams(
            needs_layout_passes=False, use_tc_tiling_on_sc=True),
        scratch_types=[
            pltpu.VMEM((N_CHUNKS, CHUNK), jnp.int32),
            pltpu.VMEM((N_CHUNKS, CHUNK), jnp.int32),
            pltpu.VMEM((2, CHUNK, LINE), jnp.float32),
            pltpu.VMEM((2, CHUNK, LINE), jnp.float32),
            pltpu.VMEM((B_PER_W,), jnp.float32),
            pltpu.SemaphoreType.DMA,
        ],
    )
    return k(theta128, u3, v3)


def _tc_body(negl_ref, r_ref, t_ref, o_ref):
    w = jnp.clip(negl_ref[...], 1.0 + 1e-6, 100.0)
    duv = jnp.log(w + jnp.sqrt((w - 1.0) * (w + 1.0)))
    o_ref[...] = 1.0 / (jnp.exp((duv - r_ref[0, 0]) / t_ref[0, 0]) + 1.0)


def _tc_tail(negl2d, r2d, t2d):
    return pl.pallas_call(
        _tc_body,
        out_shape=jax.ShapeDtypeStruct(negl2d.shape, jnp.float32),
        in_specs=[
            pl.BlockSpec(memory_space=pltpu.VMEM),
            pl.BlockSpec(memory_space=pltpu.SMEM),
            pl.BlockSpec(memory_space=pltpu.SMEM),
        ],
        out_specs=pl.BlockSpec(memory_space=pltpu.VMEM),
    )(negl2d, r2d, t2d)


def kernel(u, v, theta, r, t):
    u3 = u.astype(jnp.int32).reshape(NUM_WORKERS, N_CHUNKS, CHUNK)
    v3 = v.astype(jnp.int32).reshape(NUM_WORKERS, N_CHUNKS, CHUNK)
    proj = jnp.concatenate(
        [jnp.eye(DIM, dtype=jnp.float32),
         jnp.zeros((DIM, LINE - DIM), jnp.float32)], axis=1)
    theta128 = lax.dot(theta, proj, precision=lax.Precision.HIGHEST)
    negl = _sc_lorentz(theta128, u3, v3)
    r2d = jnp.asarray(r, jnp.float32).reshape(1, 1)
    t2d = jnp.asarray(t, jnp.float32).reshape(1, 1)
    out = _tc_tail(negl.reshape(128, 128), r2d, t2d)
    return out.reshape(BATCH, 1)


# speedup vs baseline: 1.6512x; 1.6512x over previous
import functools

import jax
import jax.numpy as jnp
from jax import lax
from jax.experimental import pallas as pl
from jax.experimental.pallas import tpu as pltpu
from jax.experimental.pallas import tpu_sc as plsc

NUM_ITEMS_ = 1000000
BATCH = 16384
DIM = 32
LINE = 128
NUM_CORES = 2
NUM_SUBCORES = 16
NUM_WORKERS = NUM_CORES * NUM_SUBCORES
B_PER_W = BATCH // NUM_WORKERS
CHUNK = 128
N_CHUNKS = B_PER_W // CHUNK
LANES = 16


def _sc_body(theta_hbm, u_hbm, v_hbm, out_hbm,
             idx_u, idx_v, rows_u, rows_v, acc_v, sem):
    wid = lax.axis_index("s") * NUM_CORES + lax.axis_index("c")
    base = wid * B_PER_W

    pltpu.sync_copy(u_hbm.at[wid], idx_u)
    pltpu.sync_copy(v_hbm.at[wid], idx_v)

    def fire(c, buf):
        return [
            pltpu.async_copy(theta_hbm.at[idx_u.at[c]], rows_u.at[buf], sem),
            pltpu.async_copy(theta_hbm.at[idx_v.at[c]], rows_v.at[buf], sem),
        ]

    lane = lax.iota(jnp.int32, LANES)
    pending = fire(0, 0)
    for c in range(N_CHUNKS):
        nxt = fire(c + 1, (c + 1) % 2) if c + 1 < N_CHUNKS else []
        for cp in pending:
            cp.wait()
        pending = nxt
        bvec = jnp.full((LANES,), c % 2, jnp.int32)

        def grp(g, carry, c=c, bvec=bvec):
            rvec = g * LANES + lane
            d0 = jnp.zeros((LANES,), jnp.int32)
            acc = (plsc.load_gather(rows_u, [bvec, rvec, d0]) *
                   plsc.load_gather(rows_v, [bvec, rvec, d0]))
            for d in range(1, DIM):
                dv = jnp.full((LANES,), d, jnp.int32)
                acc = acc - (plsc.load_gather(rows_u, [bvec, rvec, dv]) *
                             plsc.load_gather(rows_v, [bvec, rvec, dv]))
            acc_v[pl.ds(c * CHUNK + g * LANES, LANES)] = acc
            return carry

        lax.fori_loop(0, CHUNK // LANES, grp, 0)

    pltpu.sync_copy(acc_v, out_hbm.at[pl.ds(base, B_PER_W)])


def _sc_lorentz(theta128, u3, v3):
    mesh = plsc.VectorSubcoreMesh(core_axis_name="c", subcore_axis_name="s")
    k = pl.kernel(
        _sc_body,
        out_type=jax.ShapeDtypeStruct((BATCH,), jnp.float32),
        mesh=mesh,
        compiler_params=pltpu.CompilerParams(
            needs_layout_passes=False, use_tc_tiling_on_sc=True),
        scratch_types=[
            pltpu.VMEM((N_CHUNKS, CHUNK), jnp.int32),
            pltpu.VMEM((N_CHUNKS, CHUNK), jnp.int32),
            pltpu.VMEM((2, CHUNK, LINE), jnp.float32),
            pltpu.VMEM((2, CHUNK, LINE), jnp.float32),
            pltpu.VMEM((B_PER_W,), jnp.float32),
            pltpu.SemaphoreType.DMA,
        ],
    )
    return k(theta128, u3, v3)


def _tc_body(negl_ref, r_ref, t_ref, o_ref):
    w = jnp.clip(negl_ref[...], 1.0 + 1e-6, 100.0)
    duv = jnp.log(w + jnp.sqrt((w - 1.0) * (w + 1.0)))
    o_ref[...] = 1.0 / (jnp.exp((duv - r_ref[0, 0]) / t_ref[0, 0]) + 1.0)


def _tc_tail(negl2d, r2d, t2d):
    return pl.pallas_call(
        _tc_body,
        out_shape=jax.ShapeDtypeStruct(negl2d.shape, jnp.float32),
        in_specs=[
            pl.BlockSpec(memory_space=pltpu.VMEM),
            pl.BlockSpec(memory_space=pltpu.SMEM),
            pl.BlockSpec(memory_space=pltpu.SMEM),
        ],
        out_specs=pl.BlockSpec(memory_space=pltpu.VMEM),
    )(negl2d, r2d, t2d)


def kernel(u, v, theta, r, t):
    u3 = u.astype(jnp.int32).reshape(NUM_WORKERS, N_CHUNKS, CHUNK)
    v3 = v.astype(jnp.int32).reshape(NUM_WORKERS, N_CHUNKS, CHUNK)
    theta128 = jnp.pad(theta, ((0, 0), (0, LINE - DIM)))
    negl = _sc_lorentz(theta128, u3, v3)
    r2d = jnp.asarray(r, jnp.float32).reshape(1, 1)
    t2d = jnp.asarray(t, jnp.float32).reshape(1, 1)
    out = _tc_tail(negl.reshape(128, 128), r2d, t2d)
    return out.reshape(BATCH, 1)
